# baseline (speedup 1.0000x reference)
import functools

import jax
import jax.numpy as jnp
from jax.experimental import pallas as pl

K_NEIGH = 20
CHANNELS = [64, 64, 128, 256]
EMB = 1024


def _topk_idx(feats):
    xt = jnp.transpose(feats, (0, 2, 1))
    inner = -2.0 * jnp.einsum('bcn,bcm->bnm', xt, xt)
    xx = jnp.sum(xt * xt, axis=1)
    pd = -xx[:, :, None] - inner - xx[:, None, :]
    return jax.lax.top_k(pd, K_NEIGH)[1]


def _gather_max(z, idx):
    g = jax.vmap(lambda zb, ib: zb[ib])(z, idx)
    return jnp.max(g, axis=2)


def _bn01(x, g, b):
    m = jnp.mean(x, axis=(0, 1), keepdims=True)
    v = jnp.var(x, axis=(0, 1), keepdims=True)
    return (x - m) / jnp.sqrt(v + 1e-5) * g + b


def _head_kernel(h_ref, W1_ref, g1_ref, b1_ref, W2_ref, b2b_ref, g2_ref,
                 b2_ref, W3_ref, b3b_ref, o_ref):
    h = h_ref[...]
    z = jnp.dot(h, W1_ref[...].T, preferred_element_type=jnp.float32)
    m = jnp.mean(z, axis=0, keepdims=True)
    v = jnp.mean((z - m) ** 2, axis=0, keepdims=True)
    z = (z - m) * jax.lax.rsqrt(v + 1e-5) * g1_ref[...] + b1_ref[...]
    z = jnp.where(z >= 0, z, 0.2 * z)
    z2 = jnp.dot(z, W2_ref[...].T, preferred_element_type=jnp.float32) + b2b_ref[...]
    m2 = jnp.mean(z2, axis=0, keepdims=True)
    v2 = jnp.mean((z2 - m2) ** 2, axis=0, keepdims=True)
    z2 = (z2 - m2) * jax.lax.rsqrt(v2 + 1e-5) * g2_ref[...] + b2_ref[...]
    z2 = jnp.where(z2 >= 0, z2, 0.2 * z2)
    o_ref[...] = jnp.dot(z2, W3_ref[...].T, preferred_element_type=jnp.float32) + b3b_ref[...]


def _head(h, params):
    r = lambda a: a.reshape(1, -1)
    return pl.pallas_call(
        _head_kernel,
        out_shape=jax.ShapeDtypeStruct((h.shape[0], 40), jnp.float32),
    )(h, params['W1'], r(params['g1']), r(params['b1']),
      params['W2'], r(params['b2b']), r(params['g2']), r(params['b2']),
      params['W3'], r(params['b3b']))


def kernel_verbatim(x, ops, params):
    del ops
    x0 = jnp.transpose(x, (0, 2, 1))
    def knn_t(feats):
        xt = jnp.transpose(feats, (0, 2, 1))
        inner = -2.0 * jnp.einsum('bcn,bcm->bnm', xt, xt)
        xx = jnp.sum(xt * xt, axis=1)
        pd = -xx[:, :, None] - inner - xx[:, None, :]
        return jax.lax.top_k(pd, K_NEIGH)[1]
    def ggf(xf):
        idx = knn_t(xf)
        feat = jax.vmap(lambda xb, ib: xb[ib])(xf, idx)
        xc = jnp.broadcast_to(xf[:, :, None, :], feat.shape)
        return jnp.concatenate([feat - xc, xc], axis=3)
    def ggf4(xf):
        idx = knn_t(xf[..., 0:3])
        feat = jax.vmap(lambda xb, ib: xb[ib])(xf, idx)
        xc = jnp.broadcast_to(xf[:, :, None, :], feat.shape)
        return jnp.concatenate([feat[..., 0:3] - xc[..., 0:3], xc], axis=3)
    def conv0(feat, W, g, b):
        y = jnp.einsum('bnkc,oc->bnko', feat, W)
        y = jnp.max(y, axis=2)
        return jax.nn.relu(_bn01(y, g, b))
    inputs = [x0]
    p = ggf(x0[..., 0:3])
    for layer in range(4):
        feat = ggf4(inputs[layer]) if layer == 0 else ggf(inputs[layer])
        _x = conv0(feat, params['Wx%d' % layer], params['gx%d' % layer], params['bx%d' % layer])
        _W = conv0(p, params['Ww%d' % layer], params['gw%d' % layer], params['bw%d' % layer])
        inputs.append(_x * _W)
    xc = jnp.concatenate(inputs[1:], axis=2)
    y = xc @ params['Wp'].T
    y = _bn01(y, params['gp'], params['bp'])
    y = jnp.where(y >= 0, y, 0.2 * y)
    h = jnp.concatenate([jnp.max(y, axis=1), jnp.mean(y, axis=1)], axis=1)
    return _head(h, params)


def kernel(x, ops, params):
    del ops
    x0 = jnp.transpose(x, (0, 2, 1))
    coords = x0[..., 0:3]
    idx0 = _topk_idx(coords)

    cur = x0
    idx = idx0
    outs = []
    for l in range(4):
        Wx = params['Wx%d' % l]
        Ww = params['Ww%d' % l]
        if l == 0:
            A = Wx[:, 0:3]
            Bm = Wx[:, 3:9]
            zx = coords @ A.T
            const = x0 @ Bm.T - zx
        else:
            C = cur.shape[-1]
            A = Wx[:, 0:C]
            Bm = Wx[:, C:]
            zx = cur @ A.T
            const = cur @ (Bm - A).T
            idx = _topk_idx(cur)
        _x_pre = _gather_max(zx, idx) + const

        P = Ww[:, 0:3]
        zw = coords @ P.T
        constw = coords @ (Ww[:, 3:6] - P).T
        _W_pre = _gather_max(zw, idx0) + constw

        _xa = jax.nn.relu(_bn01(_x_pre, params['gx%d' % l], params['bx%d' % l]))
        _Wa = jax.nn.relu(_bn01(_W_pre, params['gw%d' % l], params['bw%d' % l]))
        cur = _xa * _Wa
        outs.append(cur)

    xc = jnp.concatenate(outs, axis=2)
    y = xc @ params['Wp'].T
    y = _bn01(y, params['gp'], params['bp'])
    y = jnp.where(y >= 0, y, 0.2 * y)
    h = jnp.concatenate([jnp.max(y, axis=1), jnp.mean(y, axis=1)], axis=1)
    return _head(h, params)


# speedup vs baseline: 15.6623x; 15.6623x over previous
import functools

import jax
import jax.numpy as jnp
from jax import lax
from jax.experimental import pallas as pl
from jax.experimental.pallas import tpu as pltpu
from jax.experimental.pallas import tpu_sc as plsc

K_NEIGH = 20
N_PTS = 1024
B_SZ = 8
R_TOT = B_SZ * N_PTS
T_TOT = K_NEIGH * R_TOT


def _bf(a):
    return a.astype(jnp.bfloat16)


def _dot_t(a, b):
    return lax.dot_general(a, b, (((1,), (1,)), ((), ())),
                           preferred_element_type=jnp.float32)


def _knn_kernel(f_ref, o_ref):
    b = pl.program_id(0)
    xb = f_ref[0]
    inner = -2.0 * _dot_t(_bf(xb), _bf(xb))
    xx = jnp.sum(xb * xb, axis=1, keepdims=True)
    pd = (-xx) - inner - jnp.transpose(xx)

    iota_col = lax.broadcasted_iota(
        jnp.int32, (N_PTS, N_PTS), 1).astype(jnp.float32)
    k_row = lax.broadcasted_iota(jnp.int32, (N_PTS, 32), 1)
    acc0 = jnp.zeros((N_PTS, 32), jnp.float32)

    def body(k, carry):
        keys, acc = carry
        m = jnp.max(keys, axis=1, keepdims=True)
        eq = keys == m
        cand = jnp.where(eq, iota_col, jnp.float32(N_PTS))
        first = jnp.min(cand, axis=1, keepdims=True)
        acc = jnp.where(k_row == k, first, acc)
        keys = jnp.where(iota_col == first, -jnp.inf, keys)
        return keys, acc

    _, acc = lax.fori_loop(0, K_NEIGH, body, (pd, acc0))
    t = jnp.transpose(acc)
    o_ref[...] = t[0:K_NEIGH].astype(jnp.int32) + b * N_PTS


def _knn_pallas(feats):
    B, N, C = feats.shape
    return pl.pallas_call(
        _knn_kernel,
        grid=(B,),
        in_specs=[pl.BlockSpec((1, N, C), lambda b: (b, 0, 0))],
        out_specs=pl.BlockSpec((K_NEIGH, N), lambda b: (0, b)),
        out_shape=jax.ShapeDtypeStruct((K_NEIGH, B * N), jnp.int32),
    )(feats)


def _sc_gather(table, idx):
    R, C = table.shape
    T = idx.shape[0]
    NW = 32
    per = T // NW
    CH = 128
    mesh = plsc.VectorSubcoreMesh(core_axis_name="c", subcore_axis_name="s")

    @functools.partial(
        pl.kernel, mesh=mesh,
        out_type=jax.ShapeDtypeStruct((T, C), jnp.float32),
        scratch_types=[pltpu.VMEM((CH,), jnp.int32),
                       pltpu.VMEM((CH, C), jnp.float32),
                       pltpu.SemaphoreType.DMA],
    )
    def k(table_hbm, idx_hbm, out_hbm, idx_v, rows_v, sem):
        wid = lax.axis_index("s") * 2 + lax.axis_index("c")
        base = wid * per

        def body(i, _):
            off = base + i * CH
            pltpu.sync_copy(idx_hbm.at[pl.ds(off, CH)], idx_v)
            pltpu.async_copy(table_hbm.at[idx_v], rows_v, sem).wait()
            pltpu.sync_copy(rows_v, out_hbm.at[pl.ds(off, CH)])
            return 0

        lax.fori_loop(0, per // CH, body, 0)

    return k(table, idx)


def _wconv_kernel(g_ref, x_ref, w3_ref, w6_ref, pre_ref, s_ref, q_ref):
    xp = x_ref[0]
    coords = xp[:, 0:3]
    x6 = xp[:, 0:6]
    const = _dot_t(_bf(x6), _bf(w6_ref[...]))
    w3 = _bf(w3_ref[...])
    m = None
    for k in range(K_NEIGH):
        fk = _bf(g_ref[k, 0][:, 0:3] - coords)
        yk = _dot_t(fk, w3) + const
        m = yk if m is None else jnp.maximum(m, yk)
    pre_ref[0] = m
    s_ref[0] = jnp.sum(m, axis=0, keepdims=True)
    q_ref[0] = jnp.sum(m * m, axis=0, keepdims=True)


def _wconv(g0, x0p, w3, w6):
    return pl.pallas_call(
        _wconv_kernel,
        grid=(B_SZ,),
        in_specs=[
            pl.BlockSpec((K_NEIGH, 1, N_PTS, 128), lambda b: (0, b, 0, 0)),
            pl.BlockSpec((1, N_PTS, 8), lambda b: (b, 0, 0)),
            pl.BlockSpec((576, 3), lambda b: (0, 0)),
            pl.BlockSpec((576, 6), lambda b: (0, 0)),
        ],
        out_specs=[
            pl.BlockSpec((1, N_PTS, 576), lambda b: (b, 0, 0)),
            pl.BlockSpec((1, 1, 576), lambda b: (b, 0, 0)),
            pl.BlockSpec((1, 1, 576), lambda b: (b, 0, 0)),
        ],
        out_shape=[
            jax.ShapeDtypeStruct((B_SZ, N_PTS, 576), jnp.float32),
            jax.ShapeDtypeStruct((B_SZ, 1, 576), jnp.float32),
            jax.ShapeDtypeStruct((B_SZ, 1, 576), jnp.float32),
        ],
    )(g0, x0p, w3, w6)


def _xconv_kernel(g_ref, x_ref, w_ref, pre_ref, s_ref, q_ref, *, C):
    xb = x_ref[0]
    W = w_ref[...]
    const = _dot_t(_bf(xb), _bf(W[:, C:]))
    wa = _bf(W[:, 0:C])
    m = None
    for k in range(K_NEIGH):
        fk = _bf(g_ref[k, 0][:, 0:C] - xb)
        yk = _dot_t(fk, wa) + const
        m = yk if m is None else jnp.maximum(m, yk)
    pre_ref[0] = m
    s_ref[0] = jnp.sum(m, axis=0, keepdims=True)
    q_ref[0] = jnp.sum(m * m, axis=0, keepdims=True)


def _xconv(g, cur, W):
    C = cur.shape[-1]
    C2 = W.shape[0]
    return pl.pallas_call(
        functools.partial(_xconv_kernel, C=C),
        grid=(B_SZ,),
        in_specs=[
            pl.BlockSpec((K_NEIGH, 1, N_PTS, 128), lambda b: (0, b, 0, 0)),
            pl.BlockSpec((1, N_PTS, C), lambda b: (b, 0, 0)),
            pl.BlockSpec((C2, 2 * C), lambda b: (0, 0)),
        ],
        out_specs=[
            pl.BlockSpec((1, N_PTS, C2), lambda b: (b, 0, 0)),
            pl.BlockSpec((1, 1, C2), lambda b: (b, 0, 0)),
            pl.BlockSpec((1, 1, C2), lambda b: (b, 0, 0)),
        ],
        out_shape=[
            jax.ShapeDtypeStruct((B_SZ, N_PTS, C2), jnp.float32),
            jax.ShapeDtypeStruct((B_SZ, 1, C2), jnp.float32),
            jax.ShapeDtypeStruct((B_SZ, 1, C2), jnp.float32),
        ],
    )(g, cur, W)


def _apply_kernel(px_ref, pw_ref, sx_ref, qx_ref, sw_ref, qw_ref,
                  gx_ref, bx_ref, gw_ref, bw_ref, o_ref):
    cnt = jnp.float32(B_SZ * N_PTS)

    def norm(p, s_ref, q_ref, g_ref, b_ref):
        s = jnp.sum(s_ref[:, 0, :], axis=0, keepdims=True)
        q = jnp.sum(q_ref[:, 0, :], axis=0, keepdims=True)
        m = s / cnt
        v = q / cnt - m * m
        z = (p - m) / jnp.sqrt(v + 1e-5) * g_ref[...] + b_ref[...]
        return jnp.maximum(z, 0.0)

    o_ref[0] = (norm(px_ref[0], sx_ref, qx_ref, gx_ref, bx_ref) *
                norm(pw_ref[0], sw_ref, qw_ref, gw_ref, bw_ref))


def _apply(px, pw, sx, qx, sw, qw, g1, b1, g2, b2):
    C2 = px.shape[-1]
    r = lambda a: a.reshape(1, -1)
    return pl.pallas_call(
        _apply_kernel,
        grid=(B_SZ,),
        in_specs=[
            pl.BlockSpec((1, N_PTS, C2), lambda b: (b, 0, 0)),
            pl.BlockSpec((1, N_PTS, C2), lambda b: (b, 0, 0)),
            pl.BlockSpec((B_SZ, 1, C2), lambda b: (0, 0, 0)),
            pl.BlockSpec((B_SZ, 1, C2), lambda b: (0, 0, 0)),
            pl.BlockSpec((B_SZ, 1, C2), lambda b: (0, 0, 0)),
            pl.BlockSpec((B_SZ, 1, C2), lambda b: (0, 0, 0)),
            pl.BlockSpec((1, C2), lambda b: (0, 0)),
            pl.BlockSpec((1, C2), lambda b: (0, 0)),
            pl.BlockSpec((1, C2), lambda b: (0, 0)),
            pl.BlockSpec((1, C2), lambda b: (0, 0)),
        ],
        out_specs=pl.BlockSpec((1, N_PTS, C2), lambda b: (b, 0, 0)),
        out_shape=jax.ShapeDtypeStruct((B_SZ, N_PTS, C2), jnp.float32),
    )(px, pw, sx, qx, sw, qw, r(g1), r(b1), r(g2), r(b2))


def _proj_kernel(c1_ref, c2_ref, c3_ref, c4_ref, wp_ref, y_ref, s_ref, q_ref):
    wp = wp_ref[...]
    y = _dot_t(_bf(c1_ref[0]), _bf(wp[:, 0:64]))
    y = y + _dot_t(_bf(c2_ref[0]), _bf(wp[:, 64:128]))
    y = y + _dot_t(_bf(c3_ref[0]), _bf(wp[:, 128:256]))
    y = y + _dot_t(_bf(c4_ref[0]), _bf(wp[:, 256:512]))
    y_ref[0] = y
    s_ref[0] = jnp.sum(y, axis=0, keepdims=True)
    q_ref[0] = jnp.sum(y * y, axis=0, keepdims=True)


def _proj(c1, c2, c3, c4, Wp):
    return pl.pallas_call(
        _proj_kernel,
        grid=(B_SZ,),
        in_specs=[
            pl.BlockSpec((1, N_PTS, 64), lambda b: (b, 0, 0)),
            pl.BlockSpec((1, N_PTS, 64), lambda b: (b, 0, 0)),
            pl.BlockSpec((1, N_PTS, 128), lambda b: (b, 0, 0)),
            pl.BlockSpec((1, N_PTS, 256), lambda b: (b, 0, 0)),
            pl.BlockSpec((1024, 512), lambda b: (0, 0)),
        ],
        out_specs=[
            pl.BlockSpec((1, N_PTS, 1024), lambda b: (b, 0, 0)),
            pl.BlockSpec((1, 1, 1024), lambda b: (b, 0, 0)),
            pl.BlockSpec((1, 1, 1024), lambda b: (b, 0, 0)),
        ],
        out_shape=[
            jax.ShapeDtypeStruct((B_SZ, N_PTS, 1024), jnp.float32),
            jax.ShapeDtypeStruct((B_SZ, 1, 1024), jnp.float32),
            jax.ShapeDtypeStruct((B_SZ, 1, 1024), jnp.float32),
        ],
    )(c1, c2, c3, c4, Wp)


def _pool_kernel(y_ref, s_ref, q_ref, gp_ref, bp_ref, mx_ref, mn_ref):
    cnt = jnp.float32(B_SZ * N_PTS)
    s = jnp.sum(s_ref[:, 0, :], axis=0, keepdims=True)
    q = jnp.sum(q_ref[:, 0, :], axis=0, keepdims=True)
    m = s / cnt
    v = q / cnt - m * m
    z = (y_ref[0] - m) / jnp.sqrt(v + 1e-5) * gp_ref[...] + bp_ref[...]
    z = jnp.where(z >= 0, z, 0.2 * z)
    mx_ref[0] = jnp.max(z, axis=0, keepdims=True)
    mn_ref[0] = jnp.sum(z, axis=0, keepdims=True) / jnp.float32(N_PTS)


def _pool(y, s, q, gp, bp):
    r = lambda a: a.reshape(1, -1)
    return pl.pallas_call(
        _pool_kernel,
        grid=(B_SZ,),
        in_specs=[
            pl.BlockSpec((1, N_PTS, 1024), lambda b: (b, 0, 0)),
            pl.BlockSpec((B_SZ, 1, 1024), lambda b: (0, 0, 0)),
            pl.BlockSpec((B_SZ, 1, 1024), lambda b: (0, 0, 0)),
            pl.BlockSpec((1, 1024), lambda b: (0, 0)),
            pl.BlockSpec((1, 1024), lambda b: (0, 0)),
        ],
        out_specs=[
            pl.BlockSpec((1, 1, 1024), lambda b: (b, 0, 0)),
            pl.BlockSpec((1, 1, 1024), lambda b: (b, 0, 0)),
        ],
        out_shape=[
            jax.ShapeDtypeStruct((B_SZ, 1, 1024), jnp.float32),
            jax.ShapeDtypeStruct((B_SZ, 1, 1024), jnp.float32),
        ],
    )(y, s, q, r(gp), r(bp))


def _head_kernel(h_ref, W1_ref, g1_ref, b1_ref, W2_ref, b2b_ref, g2_ref,
                 b2_ref, W3_ref, b3b_ref, o_ref):
    def bdot(a, w):
        return lax.dot_general(_bf(a), _bf(w), (((1,), (1,)), ((), ())),
                               preferred_element_type=jnp.float32)

    def bnorm(z, g, b):
        m = jnp.mean(z, axis=0, keepdims=True)
        v = jnp.mean((z - m) ** 2, axis=0, keepdims=True)
        return (z - m) / jnp.sqrt(v + 1e-5) * g + b

    h = h_ref[...]
    z = bnorm(bdot(h, W1_ref[...]), g1_ref[...], b1_ref[...])
    z = jnp.where(z >= 0, z, 0.2 * z)
    z2 = bnorm(bdot(z, W2_ref[...]) + b2b_ref[...], g2_ref[...], b2_ref[...])
    z2 = jnp.where(z2 >= 0, z2, 0.2 * z2)
    o_ref[...] = bdot(z2, W3_ref[...]) + b3b_ref[...]


def _head(h, params):
    r = lambda a: a.reshape(1, -1)
    return pl.pallas_call(
        _head_kernel,
        out_shape=jax.ShapeDtypeStruct((h.shape[0], 40), jnp.float32),
    )(h, params['W1'], r(params['g1']), r(params['b1']),
      params['W2'], r(params['b2b']), r(params['g2']), r(params['b2']),
      params['W3'], r(params['b3b']))


def kernel(x, ops, params):
    del ops
    x0 = jnp.transpose(x, (0, 2, 1))
    coords = x0[..., 0:3]
    x0p = jnp.pad(x0, ((0, 0), (0, 0), (0, 2)))

    idx0 = _knn_pallas(coords)
    t0 = jnp.pad(x0p.reshape(R_TOT, 8), ((0, 0), (0, 120)))
    g0 = _sc_gather(t0, idx0.reshape(-1))
    g0 = g0.reshape(K_NEIGH, B_SZ, N_PTS, 128)

    w3s = [params['Wx0'][:, 0:3]] + [params['Ww%d' % l][:, 0:3] for l in range(4)]
    w6s = [params['Wx0'][:, 3:9]] + [
        jnp.pad(params['Ww%d' % l][:, 3:6], ((0, 0), (0, 3))) for l in range(4)]
    w3 = jnp.concatenate(w3s, axis=0)
    w6 = jnp.concatenate(w6s, axis=0)

    pre_cat, s_cat, q_cat = _wconv(g0, x0p, w3, w6)
    off = [0, 64, 128, 192, 320, 576]
    pre_x0 = pre_cat[..., off[0]:off[1]]
    pre_w = [pre_cat[..., off[i + 1]:off[i + 2]] for i in range(4)]
    s_x0, q_x0 = s_cat[..., off[0]:off[1]], q_cat[..., off[0]:off[1]]
    s_w = [s_cat[..., off[i + 1]:off[i + 2]] for i in range(4)]
    q_w = [q_cat[..., off[i + 1]:off[i + 2]] for i in range(4)]

    cur = _apply(pre_x0, pre_w[0], s_x0, q_x0, s_w[0], q_w[0],
                 params['gx0'], params['bx0'], params['gw0'], params['bw0'])
    outs = [cur]
    for l in range(1, 4):
        C = cur.shape[-1]
        idx = _knn_pallas(cur)
        tb = cur.reshape(R_TOT, C)
        if C < 128:
            tb = jnp.pad(tb, ((0, 0), (0, 128 - C)))
        g = _sc_gather(tb, idx.reshape(-1))
        g = g.reshape(K_NEIGH, B_SZ, N_PTS, 128)
        pre_x, s_x, q_x = _xconv(g, cur, params['Wx%d' % l])
        cur = _apply(pre_x, pre_w[l], s_x, q_x, s_w[l], q_w[l],
                     params['gx%d' % l], params['bx%d' % l],
                     params['gw%d' % l], params['bw%d' % l])
        outs.append(cur)

    y, ys, yq = _proj(outs[0], outs[1], outs[2], outs[3], params['Wp'])
    hmax, hmean = _pool(y, ys, yq, params['gp'], params['bp'])
    h = jnp.concatenate([hmax.reshape(B_SZ, -1), hmean.reshape(B_SZ, -1)],
                        axis=1)
    return _head(h, params)


# speedup vs baseline: 16.7835x; 1.0716x over previous
import functools

import jax
import jax.numpy as jnp
from jax import lax
from jax.experimental import pallas as pl
from jax.experimental.pallas import tpu as pltpu
from jax.experimental.pallas import tpu_sc as plsc

K_NEIGH = 20
N_PTS = 1024
B_SZ = 8
R_TOT = B_SZ * N_PTS
T_TOT = K_NEIGH * R_TOT


def _bf(a):
    return a.astype(jnp.bfloat16)


def _dot_t(a, b):
    return lax.dot_general(a, b, (((1,), (1,)), ((), ())),
                           preferred_element_type=jnp.float32)


def _knn_kernel(f_ref, o_ref):
    b = pl.program_id(0)
    xb = f_ref[0]
    inner = -2.0 * _dot_t(_bf(xb), _bf(xb))
    xx = jnp.sum(xb * xb, axis=1, keepdims=True)
    pd = (-xx) - inner - jnp.transpose(xx)

    iota_col = lax.broadcasted_iota(
        jnp.int32, (N_PTS, N_PTS), 1).astype(jnp.float32)
    k_row = lax.broadcasted_iota(jnp.int32, (N_PTS, 32), 1)
    acc0 = jnp.zeros((N_PTS, 32), jnp.float32)

    def body(k, carry):
        keys, acc = carry
        m = jnp.max(keys, axis=1, keepdims=True)
        eq = keys == m
        cand = jnp.where(eq, iota_col, jnp.float32(N_PTS))
        first = jnp.min(cand, axis=1, keepdims=True)
        acc = jnp.where(k_row == k, first, acc)
        keys = jnp.where(iota_col == first, -jnp.inf, keys)
        return keys, acc

    _, acc = lax.fori_loop(0, K_NEIGH, body, (pd, acc0))
    t = jnp.transpose(acc)
    o_ref[...] = t[0:K_NEIGH].astype(jnp.int32) + b * N_PTS


def _knn_pallas(feats):
    B, N, C = feats.shape
    return pl.pallas_call(
        _knn_kernel,
        grid=(B,),
        in_specs=[pl.BlockSpec((1, N, C), lambda b: (b, 0, 0))],
        out_specs=pl.BlockSpec((K_NEIGH, N), lambda b: (0, b)),
        out_shape=jax.ShapeDtypeStruct((K_NEIGH, B * N), jnp.int32),
    )(feats)


def _sc_gather(table, idx):
    R, C = table.shape
    T = idx.shape[0]
    NW = 32
    per = T // NW
    CH = 512
    mesh = plsc.VectorSubcoreMesh(core_axis_name="c", subcore_axis_name="s")

    @functools.partial(
        pl.kernel, mesh=mesh,
        out_type=jax.ShapeDtypeStruct((T, C), jnp.float32),
        scratch_types=[pltpu.VMEM((CH,), jnp.int32),
                       pltpu.VMEM((CH, C), jnp.float32),
                       pltpu.SemaphoreType.DMA],
    )
    def k(table_hbm, idx_hbm, out_hbm, idx_v, rows_v, sem):
        wid = lax.axis_index("s") * 2 + lax.axis_index("c")
        base = wid * per

        def body(i, _):
            off = base + i * CH
            pltpu.sync_copy(idx_hbm.at[pl.ds(off, CH)], idx_v)
            pltpu.async_copy(table_hbm.at[idx_v], rows_v, sem).wait()
            pltpu.sync_copy(rows_v, out_hbm.at[pl.ds(off, CH)])
            return 0

        lax.fori_loop(0, per // CH, body, 0)

    return k(table, idx)


def _wconv_kernel(g_ref, x_ref, w3_ref, w6_ref, pre_ref, s_ref, q_ref):
    xp = x_ref[0]
    coords = xp[:, 0:3]
    x6 = xp[:, 0:6]
    const = _dot_t(_bf(x6), _bf(w6_ref[...]))
    w3 = _bf(w3_ref[...])
    m = None
    for k in range(K_NEIGH):
        fk = _bf(g_ref[k, 0][:, 0:3] - coords)
        yk = _dot_t(fk, w3) + const
        m = yk if m is None else jnp.maximum(m, yk)
    pre_ref[0] = m
    s_ref[0] = jnp.sum(m, axis=0, keepdims=True)
    q_ref[0] = jnp.sum(m * m, axis=0, keepdims=True)


def _wconv(g0, x0p, w3, w6):
    return pl.pallas_call(
        _wconv_kernel,
        grid=(B_SZ,),
        in_specs=[
            pl.BlockSpec((K_NEIGH, 1, N_PTS, 128), lambda b: (0, b, 0, 0)),
            pl.BlockSpec((1, N_PTS, 8), lambda b: (b, 0, 0)),
            pl.BlockSpec((576, 3), lambda b: (0, 0)),
            pl.BlockSpec((576, 6), lambda b: (0, 0)),
        ],
        out_specs=[
            pl.BlockSpec((1, N_PTS, 576), lambda b: (b, 0, 0)),
            pl.BlockSpec((1, 1, 576), lambda b: (b, 0, 0)),
            pl.BlockSpec((1, 1, 576), lambda b: (b, 0, 0)),
        ],
        out_shape=[
            jax.ShapeDtypeStruct((B_SZ, N_PTS, 576), jnp.float32),
            jax.ShapeDtypeStruct((B_SZ, 1, 576), jnp.float32),
            jax.ShapeDtypeStruct((B_SZ, 1, 576), jnp.float32),
        ],
    )(g0, x0p, w3, w6)


def _xconv_kernel(g_ref, x_ref, w_ref, pre_ref, s_ref, q_ref, *, C):
    xb = x_ref[0]
    W = w_ref[...]
    const = _dot_t(_bf(xb), _bf(W[:, C:]))
    wa = _bf(W[:, 0:C])
    m = None
    for k in range(K_NEIGH):
        fk = _bf(g_ref[k, 0][:, 0:C] - xb)
        yk = _dot_t(fk, wa) + const
        m = yk if m is None else jnp.maximum(m, yk)
    pre_ref[0] = m
    s_ref[0] = jnp.sum(m, axis=0, keepdims=True)
    q_ref[0] = jnp.sum(m * m, axis=0, keepdims=True)


def _xconv(g, cur, W):
    C = cur.shape[-1]
    C2 = W.shape[0]
    return pl.pallas_call(
        functools.partial(_xconv_kernel, C=C),
        grid=(B_SZ,),
        in_specs=[
            pl.BlockSpec((K_NEIGH, 1, N_PTS, 128), lambda b: (0, b, 0, 0)),
            pl.BlockSpec((1, N_PTS, C), lambda b: (b, 0, 0)),
            pl.BlockSpec((C2, 2 * C), lambda b: (0, 0)),
        ],
        out_specs=[
            pl.BlockSpec((1, N_PTS, C2), lambda b: (b, 0, 0)),
            pl.BlockSpec((1, 1, C2), lambda b: (b, 0, 0)),
            pl.BlockSpec((1, 1, C2), lambda b: (b, 0, 0)),
        ],
        out_shape=[
            jax.ShapeDtypeStruct((B_SZ, N_PTS, C2), jnp.float32),
            jax.ShapeDtypeStruct((B_SZ, 1, C2), jnp.float32),
            jax.ShapeDtypeStruct((B_SZ, 1, C2), jnp.float32),
        ],
    )(g, cur, W)


def _apply_kernel(px_ref, pw_ref, sx_ref, qx_ref, sw_ref, qw_ref,
                  gx_ref, bx_ref, gw_ref, bw_ref, o_ref):
    cnt = jnp.float32(B_SZ * N_PTS)

    def norm(p, s_ref, q_ref, g_ref, b_ref):
        s = jnp.sum(s_ref[:, 0, :], axis=0, keepdims=True)
        q = jnp.sum(q_ref[:, 0, :], axis=0, keepdims=True)
        m = s / cnt
        v = q / cnt - m * m
        z = (p - m) / jnp.sqrt(v + 1e-5) * g_ref[...] + b_ref[...]
        return jnp.maximum(z, 0.0)

    o_ref[0] = (norm(px_ref[0], sx_ref, qx_ref, gx_ref, bx_ref) *
                norm(pw_ref[0], sw_ref, qw_ref, gw_ref, bw_ref))


def _apply(px, pw, sx, qx, sw, qw, g1, b1, g2, b2):
    C2 = px.shape[-1]
    r = lambda a: a.reshape(1, -1)
    return pl.pallas_call(
        _apply_kernel,
        grid=(B_SZ,),
        in_specs=[
            pl.BlockSpec((1, N_PTS, C2), lambda b: (b, 0, 0)),
            pl.BlockSpec((1, N_PTS, C2), lambda b: (b, 0, 0)),
            pl.BlockSpec((B_SZ, 1, C2), lambda b: (0, 0, 0)),
            pl.BlockSpec((B_SZ, 1, C2), lambda b: (0, 0, 0)),
            pl.BlockSpec((B_SZ, 1, C2), lambda b: (0, 0, 0)),
            pl.BlockSpec((B_SZ, 1, C2), lambda b: (0, 0, 0)),
            pl.BlockSpec((1, C2), lambda b: (0, 0)),
            pl.BlockSpec((1, C2), lambda b: (0, 0)),
            pl.BlockSpec((1, C2), lambda b: (0, 0)),
            pl.BlockSpec((1, C2), lambda b: (0, 0)),
        ],
        out_specs=pl.BlockSpec((1, N_PTS, C2), lambda b: (b, 0, 0)),
        out_shape=jax.ShapeDtypeStruct((B_SZ, N_PTS, C2), jnp.float32),
    )(px, pw, sx, qx, sw, qw, r(g1), r(b1), r(g2), r(b2))


def _proj_kernel(c1_ref, c2_ref, c3_ref, c4_ref, wp_ref, y_ref, s_ref, q_ref):
    wp = wp_ref[...]
    y = _dot_t(_bf(c1_ref[0]), _bf(wp[:, 0:64]))
    y = y + _dot_t(_bf(c2_ref[0]), _bf(wp[:, 64:128]))
    y = y + _dot_t(_bf(c3_ref[0]), _bf(wp[:, 128:256]))
    y = y + _dot_t(_bf(c4_ref[0]), _bf(wp[:, 256:512]))
    y_ref[0] = y
    s_ref[0] = jnp.sum(y, axis=0, keepdims=True)
    q_ref[0] = jnp.sum(y * y, axis=0, keepdims=True)


def _proj(c1, c2, c3, c4, Wp):
    return pl.pallas_call(
        _proj_kernel,
        grid=(B_SZ,),
        in_specs=[
            pl.BlockSpec((1, N_PTS, 64), lambda b: (b, 0, 0)),
            pl.BlockSpec((1, N_PTS, 64), lambda b: (b, 0, 0)),
            pl.BlockSpec((1, N_PTS, 128), lambda b: (b, 0, 0)),
            pl.BlockSpec((1, N_PTS, 256), lambda b: (b, 0, 0)),
            pl.BlockSpec((1024, 512), lambda b: (0, 0)),
        ],
        out_specs=[
            pl.BlockSpec((1, N_PTS, 1024), lambda b: (b, 0, 0)),
            pl.BlockSpec((1, 1, 1024), lambda b: (b, 0, 0)),
            pl.BlockSpec((1, 1, 1024), lambda b: (b, 0, 0)),
        ],
        out_shape=[
            jax.ShapeDtypeStruct((B_SZ, N_PTS, 1024), jnp.float32),
            jax.ShapeDtypeStruct((B_SZ, 1, 1024), jnp.float32),
            jax.ShapeDtypeStruct((B_SZ, 1, 1024), jnp.float32),
        ],
    )(c1, c2, c3, c4, Wp)


def _pool_kernel(y_ref, s_ref, q_ref, gp_ref, bp_ref, mx_ref, mn_ref):
    cnt = jnp.float32(B_SZ * N_PTS)
    s = jnp.sum(s_ref[:, 0, :], axis=0, keepdims=True)
    q = jnp.sum(q_ref[:, 0, :], axis=0, keepdims=True)
    m = s / cnt
    v = q / cnt - m * m
    z = (y_ref[0] - m) / jnp.sqrt(v + 1e-5) * gp_ref[...] + bp_ref[...]
    z = jnp.where(z >= 0, z, 0.2 * z)
    mx_ref[0] = jnp.max(z, axis=0, keepdims=True)
    mn_ref[0] = jnp.sum(z, axis=0, keepdims=True) / jnp.float32(N_PTS)


def _pool(y, s, q, gp, bp):
    r = lambda a: a.reshape(1, -1)
    return pl.pallas_call(
        _pool_kernel,
        grid=(B_SZ,),
        in_specs=[
            pl.BlockSpec((1, N_PTS, 1024), lambda b: (b, 0, 0)),
            pl.BlockSpec((B_SZ, 1, 1024), lambda b: (0, 0, 0)),
            pl.BlockSpec((B_SZ, 1, 1024), lambda b: (0, 0, 0)),
            pl.BlockSpec((1, 1024), lambda b: (0, 0)),
            pl.BlockSpec((1, 1024), lambda b: (0, 0)),
        ],
        out_specs=[
            pl.BlockSpec((1, 1, 1024), lambda b: (b, 0, 0)),
            pl.BlockSpec((1, 1, 1024), lambda b: (b, 0, 0)),
        ],
        out_shape=[
            jax.ShapeDtypeStruct((B_SZ, 1, 1024), jnp.float32),
            jax.ShapeDtypeStruct((B_SZ, 1, 1024), jnp.float32),
        ],
    )(y, s, q, r(gp), r(bp))


def _head_kernel(h_ref, W1_ref, g1_ref, b1_ref, W2_ref, b2b_ref, g2_ref,
                 b2_ref, W3_ref, b3b_ref, o_ref):
    def bdot(a, w):
        return lax.dot_general(_bf(a), _bf(w), (((1,), (1,)), ((), ())),
                               preferred_element_type=jnp.float32)

    def bnorm(z, g, b):
        m = jnp.mean(z, axis=0, keepdims=True)
        v = jnp.mean((z - m) ** 2, axis=0, keepdims=True)
        return (z - m) / jnp.sqrt(v + 1e-5) * g + b

    h = h_ref[...]
    z = bnorm(bdot(h, W1_ref[...]), g1_ref[...], b1_ref[...])
    z = jnp.where(z >= 0, z, 0.2 * z)
    z2 = bnorm(bdot(z, W2_ref[...]) + b2b_ref[...], g2_ref[...], b2_ref[...])
    z2 = jnp.where(z2 >= 0, z2, 0.2 * z2)
    o_ref[...] = bdot(z2, W3_ref[...]) + b3b_ref[...]


def _head(h, params):
    r = lambda a: a.reshape(1, -1)
    return pl.pallas_call(
        _head_kernel,
        out_shape=jax.ShapeDtypeStruct((h.shape[0], 40), jnp.float32),
    )(h, params['W1'], r(params['g1']), r(params['b1']),
      params['W2'], r(params['b2b']), r(params['g2']), r(params['b2']),
      params['W3'], r(params['b3b']))


def kernel(x, ops, params):
    del ops
    x0 = jnp.transpose(x, (0, 2, 1))
    coords = x0[..., 0:3]
    x0p = jnp.pad(x0, ((0, 0), (0, 0), (0, 2)))

    idx0 = _knn_pallas(coords)
    t0 = jnp.pad(x0p.reshape(R_TOT, 8), ((0, 0), (0, 120)))
    g0 = _sc_gather(t0, idx0.reshape(-1))
    g0 = g0.reshape(K_NEIGH, B_SZ, N_PTS, 128)

    w3s = [params['Wx0'][:, 0:3]] + [params['Ww%d' % l][:, 0:3] for l in range(4)]
    w6s = [params['Wx0'][:, 3:9]] + [
        jnp.pad(params['Ww%d' % l][:, 3:6], ((0, 0), (0, 3))) for l in range(4)]
    w3 = jnp.concatenate(w3s, axis=0)
    w6 = jnp.concatenate(w6s, axis=0)

    pre_cat, s_cat, q_cat = _wconv(g0, x0p, w3, w6)
    off = [0, 64, 128, 192, 320, 576]
    pre_x0 = pre_cat[..., off[0]:off[1]]
    pre_w = [pre_cat[..., off[i + 1]:off[i + 2]] for i in range(4)]
    s_x0, q_x0 = s_cat[..., off[0]:off[1]], q_cat[..., off[0]:off[1]]
    s_w = [s_cat[..., off[i + 1]:off[i + 2]] for i in range(4)]
    q_w = [q_cat[..., off[i + 1]:off[i + 2]] for i in range(4)]

    cur = _apply(pre_x0, pre_w[0], s_x0, q_x0, s_w[0], q_w[0],
                 params['gx0'], params['bx0'], params['gw0'], params['bw0'])
    outs = [cur]
    for l in range(1, 4):
        C = cur.shape[-1]
        idx = _knn_pallas(cur)
        tb = cur.reshape(R_TOT, C)
        if C < 128:
            tb = jnp.pad(tb, ((0, 0), (0, 128 - C)))
        g = _sc_gather(tb, idx.reshape(-1))
        g = g.reshape(K_NEIGH, B_SZ, N_PTS, 128)
        pre_x, s_x, q_x = _xconv(g, cur, params['Wx%d' % l])
        cur = _apply(pre_x, pre_w[l], s_x, q_x, s_w[l], q_w[l],
                     params['gx%d' % l], params['bx%d' % l],
                     params['gw%d' % l], params['bw%d' % l])
        outs.append(cur)

    y, ys, yq = _proj(outs[0], outs[1], outs[2], outs[3], params['Wp'])
    hmax, hmean = _pool(y, ys, yq, params['gp'], params['bp'])
    h = jnp.concatenate([hmax.reshape(B_SZ, -1), hmean.reshape(B_SZ, -1)],
                        axis=1)
    return _head(h, params)
